# Initial kernel scaffold; baseline (speedup 1.0000x reference)
import jax
import jax.numpy as jnp
from jax.experimental import pallas as pl


def kernel(images, predictions, anchors):
    raise NotImplementedError("write your pallas kernel here")


# speedup vs baseline: 8.0130x; 8.0130x over previous
import jax
import jax.numpy as jnp
from jax.experimental import pallas as pl

_CONF_T = 0.05
_IOU_T = 0.35
_MAXDET = 100
_N = 20000
_R, _C = 160, 128
_NPAD = _R * _C
_BIG = 2 ** 30


def _nms_body(pred_ref, anc_ref, boxes_ref, scores_ref):
    iota = (jax.lax.broadcasted_iota(jnp.int32, (_R, _C), 0) * _C
            + jax.lax.broadcasted_iota(jnp.int32, (_R, _C), 1))
    in_range = iota < _N

    logit = pred_ref[0]
    score = jax.nn.sigmoid(logit)
    c = [pred_ref[1 + j] * 0.1 * anc_ref[3 + j] + anc_ref[j] for j in range(3)]
    s = [jnp.exp(pred_ref[4 + j] * 0.2) * anc_ref[3 + j] for j in range(3)]

    positive = in_range
    for v in c + s:
        positive = jnp.logical_and(positive, v > 0)
    valid0 = jnp.logical_and(positive, score > _CONF_T)
    neginf = jnp.float32(-jnp.inf)
    masked0 = jnp.where(valid0, score, neginf)

    infA = [c[j] - s[j] * 0.5 for j in range(3)]
    supA = [c[j] + s[j] * 0.5 for j in range(3)]
    areaA = s[0] * s[1] * s[2]

    slot = (jax.lax.broadcasted_iota(jnp.int32, (8, 128), 0) * 128
            + jax.lax.broadcasted_iota(jnp.int32, (8, 128), 1))

    def body(i, st):
        masked, ob, osc = st
        m = jnp.max(masked)
        eqv = masked == m
        best = jnp.min(jnp.where(eqv, iota, _BIG))
        eq1 = iota == best

        def msum(v):
            return jnp.sum(jnp.where(eq1, v, 0.0))

        bc = [msum(c[j]) for j in range(3)]
        bs = [msum(s[j]) for j in range(3)]
        bscore = msum(score)

        infB = [bc[j] - bs[j] * 0.5 for j in range(3)]
        supB = [bc[j] + bs[j] * 0.5 for j in range(3)]
        areaB = bs[0] * bs[1] * bs[2]
        ia = None
        for j in range(3):
            sup = jnp.minimum(supA[j], supB[j])
            inf = jnp.maximum(infA[j], infB[j])
            iv = jnp.maximum(0.0, sup - inf)
            ia = iv if ia is None else ia * iv
        union = jnp.maximum(areaA + areaB - ia, 1e-8)
        iou = ia / union
        masked = jnp.where(iou < _IOU_T, masked, neginf)
        masked = jnp.where(eq1, neginf, masked)

        sel = slot == i
        ob = tuple(jnp.where(sel, v, o) for v, o in zip(bc + bs, ob))
        osc = jnp.where(sel, bscore, osc)
        return masked, ob, osc

    ob0 = tuple(jnp.zeros((8, 128), jnp.float32) for _ in range(6))
    osc0 = jnp.full((8, 128), -1.0, jnp.float32)
    _, ob, osc = jax.lax.fori_loop(0, _MAXDET, body, (masked0, ob0, osc0))

    def sbody(t, st):
        rem, outb, outs = st
        m = jnp.max(rem)
        eqv = rem == m
        bidx = jnp.min(jnp.where(eqv, slot, _BIG))
        eq1 = slot == bidx
        wsel = slot == t
        outb = tuple(
            jnp.where(wsel, jnp.sum(jnp.where(eq1, o, 0.0)), ot)
            for o, ot in zip(ob, outb))
        outs = jnp.where(wsel, m, outs)
        rem = jnp.where(eq1, -1.0, rem)
        return rem, outb, outs

    outb0 = tuple(jnp.zeros((8, 128), jnp.float32) for _ in range(6))
    outs0 = jnp.zeros((8, 128), jnp.float32)
    _, outb, outs = jax.lax.fori_loop(0, _MAXDET, sbody, (osc, outb0, outs0))

    for j in range(6):
        boxes_ref[j] = outb[j]
    scores_ref[...] = outs


def kernel(images, predictions, anchors):
    pred = predictions[0]
    pred_t = jnp.pad(pred, ((0, _NPAD - _N), (0, 0))).T.reshape(7, _R, _C)
    anc_t = jnp.pad(anchors, ((0, _NPAD - _N), (0, 0))).T.reshape(6, _R, _C)
    boxes_t, scores_o = pl.pallas_call(
        _nms_body,
        out_shape=(jax.ShapeDtypeStruct((6, 8, 128), jnp.float32),
                   jax.ShapeDtypeStruct((8, 128), jnp.float32)),
    )(pred_t, anc_t)
    boxes = boxes_t.reshape(6, 8 * 128)[:, :_MAXDET].T
    scores = scores_o.reshape(8 * 128)[:_MAXDET]
    labels = jnp.zeros((_MAXDET,), jnp.float32)
    return boxes, scores, labels
